# baseline (speedup 1.0000x reference)
import functools

import jax
import jax.numpy as jnp
from jax import lax
from jax.experimental import pallas as pl
from jax.experimental.pallas import tpu as pltpu
from jax.experimental.pallas import tpu_sc as plsc

B = 16384
D = 64
DP = 128
V = 1000000
NBLK = 7813
VP = NBLK * 128
NNEG = 10
NW = 32
BPW = B // NW
CHUNK = 64
NCHUNK = BPW // CHUNK
LANES = 16
GROUPS = CHUNK // LANES
TRIPS = 123


def _sc_convert(tt):
    mesh = plsc.VectorSubcoreMesh(core_axis_name="c", subcore_axis_name="s")

    @functools.partial(
        pl.kernel,
        mesh=mesh,
        out_type=jax.ShapeDtypeStruct((VP, DP), jnp.float32),
        scratch_types=[
            pltpu.VMEM((D, 128), jnp.float32),
            pltpu.VMEM((D, 128), jnp.float32),
            pltpu.VMEM((128, 129), jnp.float32),
            pltpu.VMEM((128, 129), jnp.float32),
            pltpu.SemaphoreType.DMA,
            pltpu.SemaphoreType.DMA,
            pltpu.SemaphoreType.DMA,
            pltpu.SemaphoreType.DMA,
        ],
        compiler_params=pltpu.CompilerParams(
            needs_layout_passes=False, use_tc_tiling_on_sc=True,
            disable_bounds_checks=True),
    )
    def conv_kernel(tt_hbm, conv_hbm, inb0, inb1, outb0, outb1,
                    rsem0, rsem1, wsem0, wsem1):
        wid = lax.axis_index("s") * 2 + lax.axis_index("c")
        inbs = (inb0, inb1)
        outbs = (outb0, outb1)
        rsems = (rsem0, rsem1)
        wsems = (wsem0, wsem1)
        lanes = lax.iota(jnp.int32, LANES)

        zeros = jnp.zeros((LANES,), jnp.float32)
        def z_body(r, zcarry):
            for h in range(2):
                for j in range(D // LANES):
                    outbs[h][r, pl.ds(D + j * LANES, LANES)] = zeros
            return zcarry
        lax.fori_loop(0, 128, z_body, 0)

        for h in range(2):
            bid0 = wid + h * NW
            pltpu.async_copy(tt_hbm.at[:, pl.ds(bid0 * 128, 128)],
                             inbs[h], rsems[h])

        def trip(t, carry):
            for h in range(2):
                bid = wid + (2 * t + h) * NW
                nbid = bid + 2 * NW

                @pl.when(bid < NBLK)
                def _process():
                    @pl.when(2 * t + h >= 2)
                    def _w():
                        pltpu.make_async_copy(
                            outbs[h].at[:, pl.ds(0, DP)],
                            conv_hbm.at[pl.ds(0, 128), :], wsems[h]).wait()
                    pltpu.make_async_copy(
                        tt_hbm.at[:, pl.ds(bid * 128, 128)], inbs[h],
                        rsems[h]).wait()

                    def tr_body(dg, tcarry):
                        for di in range(8):
                            d = dg * 8 + di
                            dfull = jnp.full((LANES,), 0, jnp.int32) + d
                            for rg in range(128 // LANES):
                                v = plsc.load_gather(
                                    inbs[h], [dfull, rg * LANES + lanes])
                                plsc.store_scatter(
                                    outbs[h], [rg * LANES + lanes, dfull], v)
                        return tcarry
                    lax.fori_loop(0, D // 8, tr_body, 0)

                    @pl.when(nbid < NBLK)
                    def _r():
                        pltpu.async_copy(
                            tt_hbm.at[:, pl.ds(nbid * 128, 128)],
                            inbs[h], rsems[h])
                    pltpu.async_copy(outbs[h].at[:, pl.ds(0, DP)],
                                     conv_hbm.at[pl.ds(bid * 128, 128), :],
                                     wsems[h])
            return carry

        lax.fori_loop(0, TRIPS, trip, 0)
        for h in range(2):
            pltpu.make_async_copy(outbs[h].at[:, pl.ds(0, DP)],
                                  conv_hbm.at[pl.ds(0, 128), :],
                                  wsems[h]).wait()

    return conv_kernel(tt)


def _sc_scores(center, context, negflat, inp, outp):
    mesh = plsc.VectorSubcoreMesh(core_axis_name="c", subcore_axis_name="s")

    @functools.partial(
        pl.kernel,
        mesh=mesh,
        out_type=(jax.ShapeDtypeStruct((B,), jnp.float32),
                  jax.ShapeDtypeStruct((B * NNEG,), jnp.float32)),
        scratch_types=[
            pltpu.VMEM((CHUNK,), jnp.int32),
            pltpu.VMEM((CHUNK,), jnp.int32),
            pltpu.VMEM((CHUNK * NNEG,), jnp.int32),
            pltpu.VMEM((CHUNK, DP), jnp.float32),
            pltpu.VMEM((CHUNK, DP), jnp.float32),
            pltpu.VMEM((CHUNK * NNEG, DP), jnp.float32),
            pltpu.VMEM((CHUNK,), jnp.float32),
            pltpu.VMEM((CHUNK * NNEG,), jnp.float32),
            pltpu.SemaphoreType.DMA,
        ],
        compiler_params=pltpu.CompilerParams(
            needs_layout_passes=False, use_tc_tiling_on_sc=True),
    )
    def sc_kernel(center_hbm, context_hbm, neg_hbm, inemb_hbm, outemb_hbm,
                  pos_hbm, negsc_hbm,
                  cidx_v, oidx_v, nidx_v, crow_v, orow_v, nrow_v,
                  psc_v, nsc_v, sem):
        wid = lax.axis_index("s") * 2 + lax.axis_index("c")
        base = wid * BPW

        def chunk_body(ci, carry):
            start = base + ci * CHUNK
            pltpu.sync_copy(center_hbm.at[pl.ds(start, CHUNK)], cidx_v)
            pltpu.sync_copy(context_hbm.at[pl.ds(start, CHUNK)], oidx_v)
            pltpu.sync_copy(neg_hbm.at[pl.ds(start * NNEG, CHUNK * NNEG)],
                            nidx_v)
            copies = [
                pltpu.async_copy(inemb_hbm.at[cidx_v], crow_v, sem),
                pltpu.async_copy(outemb_hbm.at[oidx_v], orow_v, sem),
            ]
            for j in range(NNEG):
                copies.append(pltpu.async_copy(
                    outemb_hbm.at[nidx_v.at[pl.ds(j * CHUNK, CHUNK)]],
                    nrow_v.at[pl.ds(j * CHUNK, CHUNK)], sem))
            for cp in copies:
                cp.wait()

            def group_body(t, gcarry):
                lanes = lax.iota(jnp.int32, LANES)
                ridx = t * LANES + lanes
                accp = jnp.zeros((LANES,), jnp.float32)
                accn = [jnp.zeros((LANES,), jnp.float32) for _ in range(NNEG)]
                for s in range(D):
                    didx = (lanes + s) & (D - 1)
                    cv = plsc.load_gather(crow_v, [ridx, didx])
                    ov = plsc.load_gather(orow_v, [ridx, didx])
                    accp = accp + cv * ov
                    for k in range(NNEG):
                        nv = plsc.load_gather(
                            nrow_v, [ridx * NNEG + k, didx])
                        accn[k] = accn[k] + cv * nv
                psc_v[pl.ds(t * LANES, LANES)] = accp
                for k in range(NNEG):
                    plsc.store_scatter(nsc_v, [ridx * NNEG + k], accn[k])
                return gcarry

            lax.fori_loop(0, GROUPS, group_body, 0)
            pltpu.sync_copy(psc_v, pos_hbm.at[pl.ds(start, CHUNK)])
            pltpu.sync_copy(nsc_v,
                            negsc_hbm.at[pl.ds(start * NNEG, CHUNK * NNEG)])
            return carry

        lax.fori_loop(0, NCHUNK, chunk_body, 0)

    return sc_kernel(center, context, negflat, inp, outp)


def _tc_loss(pos, neg):
    def body(p_ref, n_ref, o_ref):
        total = jnp.sum(jax.nn.log_sigmoid(p_ref[...]))
        total = total + jnp.sum(jax.nn.log_sigmoid(-n_ref[...]))
        o_ref[...] = jnp.reshape(-total / B, (1, 1))

    return pl.pallas_call(
        body,
        out_shape=jax.ShapeDtypeStruct((1, 1), jnp.float32),
    )(pos, neg)


def kernel(center, context, negatives, in_embed, out_embed):
    center = center.astype(jnp.int32)
    context = context.astype(jnp.int32)
    negflat = negatives.astype(jnp.int32).reshape(B * NNEG)
    inp = _sc_convert(jnp.swapaxes(in_embed, 0, 1))
    outp = _sc_convert(jnp.swapaxes(out_embed, 0, 1))
    pos, neg = _sc_scores(center, context, negflat, inp, outp)
    loss = _tc_loss(pos.reshape(128, B // 128),
                    neg.reshape(1280, B // 128))
    return loss[0, 0]


# speedup vs baseline: 2.1968x; 2.1968x over previous
import functools

import jax
import jax.numpy as jnp
from jax import lax
from jax.experimental import pallas as pl
from jax.experimental.pallas import tpu as pltpu
from jax.experimental.pallas import tpu_sc as plsc

B = 16384
D = 64
DP = 128
NNEG = 10
NW = 32
BPW = B // NW
CHUNK = 64
NCHUNK = BPW // CHUNK
LANES = 16
GROUPS = CHUNK // LANES


def _sc_scores(center, context, negflat, inp, outp):
    mesh = plsc.VectorSubcoreMesh(core_axis_name="c", subcore_axis_name="s")

    @functools.partial(
        pl.kernel,
        mesh=mesh,
        out_type=(jax.ShapeDtypeStruct((B,), jnp.float32),
                  jax.ShapeDtypeStruct((B * NNEG,), jnp.float32)),
        scratch_types=[
            pltpu.VMEM((CHUNK,), jnp.int32),
            pltpu.VMEM((CHUNK,), jnp.int32),
            pltpu.VMEM((CHUNK * NNEG,), jnp.int32),
            pltpu.VMEM((CHUNK, DP), jnp.float32),
            pltpu.VMEM((CHUNK, DP), jnp.float32),
            pltpu.VMEM((CHUNK * NNEG, DP), jnp.float32),
            pltpu.VMEM((CHUNK,), jnp.float32),
            pltpu.VMEM((CHUNK * NNEG,), jnp.float32),
            pltpu.SemaphoreType.DMA,
        ],
        compiler_params=pltpu.CompilerParams(
            needs_layout_passes=False, use_tc_tiling_on_sc=True),
    )
    def sc_kernel(center_hbm, context_hbm, neg_hbm, inemb_hbm, outemb_hbm,
                  pos_hbm, negsc_hbm,
                  cidx_v, oidx_v, nidx_v, crow_v, orow_v, nrow_v,
                  psc_v, nsc_v, sem):
        wid = lax.axis_index("s") * 2 + lax.axis_index("c")
        base = wid * BPW

        def chunk_body(ci, carry):
            start = base + ci * CHUNK
            pltpu.sync_copy(center_hbm.at[pl.ds(start, CHUNK)], cidx_v)
            pltpu.sync_copy(context_hbm.at[pl.ds(start, CHUNK)], oidx_v)
            pltpu.sync_copy(neg_hbm.at[pl.ds(start * NNEG, CHUNK * NNEG)],
                            nidx_v)
            copies = [
                pltpu.async_copy(inemb_hbm.at[cidx_v], crow_v, sem),
                pltpu.async_copy(outemb_hbm.at[oidx_v], orow_v, sem),
            ]
            for j in range(NNEG):
                copies.append(pltpu.async_copy(
                    outemb_hbm.at[nidx_v.at[pl.ds(j * CHUNK, CHUNK)]],
                    nrow_v.at[pl.ds(j * CHUNK, CHUNK)], sem))
            for cp in copies:
                cp.wait()

            def group_body(t, gcarry):
                lanes = lax.iota(jnp.int32, LANES)
                ridx = t * LANES + lanes
                accp = jnp.zeros((LANES,), jnp.float32)
                accn = [jnp.zeros((LANES,), jnp.float32) for _ in range(NNEG)]
                for s in range(D):
                    didx = (lanes + s) & (D - 1)
                    cv = plsc.load_gather(crow_v, [ridx, didx])
                    ov = plsc.load_gather(orow_v, [ridx, didx])
                    accp = accp + cv * ov
                    for k in range(NNEG):
                        nv = plsc.load_gather(
                            nrow_v, [ridx * NNEG + k, didx])
                        accn[k] = accn[k] + cv * nv
                psc_v[pl.ds(t * LANES, LANES)] = accp
                for k in range(NNEG):
                    plsc.store_scatter(nsc_v, [ridx * NNEG + k], accn[k])
                return gcarry

            lax.fori_loop(0, GROUPS, group_body, 0)
            pltpu.sync_copy(psc_v, pos_hbm.at[pl.ds(start, CHUNK)])
            pltpu.sync_copy(nsc_v,
                            negsc_hbm.at[pl.ds(start * NNEG, CHUNK * NNEG)])
            return carry

        lax.fori_loop(0, NCHUNK, chunk_body, 0)

    return sc_kernel(center, context, negflat, inp, outp)


def _tc_loss(pos, neg):
    def body(p_ref, n_ref, o_ref):
        total = jnp.sum(jax.nn.log_sigmoid(p_ref[...]))
        total = total + jnp.sum(jax.nn.log_sigmoid(-n_ref[...]))
        o_ref[...] = jnp.reshape(-total / B, (1, 1))

    return pl.pallas_call(
        body,
        out_shape=jax.ShapeDtypeStruct((1, 1), jnp.float32),
    )(pos, neg)


def kernel(center, context, negatives, in_embed, out_embed):
    center = center.astype(jnp.int32)
    context = context.astype(jnp.int32)
    negflat = negatives.astype(jnp.int32).reshape(B * NNEG)
    inp = jnp.pad(in_embed, ((0, 0), (0, DP - D)))
    outp = jnp.pad(out_embed, ((0, 0), (0, DP - D)))
    pos, neg = _sc_scores(center, context, negflat, inp, outp)
    loss = _tc_loss(pos.reshape(128, B // 128),
                    neg.reshape(1280, B // 128))
    return loss[0, 0]
